# baseline (speedup 1.0000x reference)
import functools

import jax
import jax.numpy as jnp
from jax import lax
from jax.experimental import pallas as pl
from jax.experimental.pallas import tpu as pltpu
from jax.experimental.pallas import tpu_sc as plsc

_TOP_P = 0.98
_B, _V = 128, 100000
_CH = 4000
_NCHUNK = _V // _CH
_CVECS = _CH // 16
_PVECS = _V // 16
_K = 1024
_ROWS_PER_W = 4


def _splat(x, dtype=jnp.float32):
    return jnp.broadcast_to(jnp.asarray(x, dtype), (16,))


def _perm(v, idx):
    dnums = lax.GatherDimensionNumbers(
        offset_dims=(), collapsed_slice_dims=(0,), start_index_map=(0,))
    return lax.gather(v, idx[:, None], dnums, (1,),
                      unique_indices=True, indices_are_sorted=False,
                      mode=lax.GatherScatterMode.PROMISE_IN_BOUNDS)


def _allsum(v):
    lane = lax.iota(jnp.int32, 16)
    for sh in (8, 4, 2, 1):
        v = v + _perm(v, lane ^ sh)
    return v


def _allmax(v):
    lane = lax.iota(jnp.int32, 16)
    for sh in (8, 4, 2, 1):
        v = jnp.maximum(v, _perm(v, lane ^ sh))
    return v


def _sqrt16(a):
    i = lax.bitcast_convert_type(a, jnp.int32)
    y = lax.bitcast_convert_type(_splat(0x5F3759DF, jnp.int32) - (i >> 1), jnp.float32)
    half = _splat(0.5)
    three_half = _splat(1.5)
    for _ in range(4):
        y = y * (three_half - half * a * y * y)
    return a * y


def _body(nl_hbm, br_hbm, dl_hbm, out_hbm, parr, hist, bufx, bufd):
    nc = 2
    wid = lax.axis_index("s") * nc + lax.axis_index("c")
    lane = lax.iota(jnp.int32, 16)
    zero16 = _splat(0.0)
    neginf = _splat(-jnp.inf)

    def sum_sq_of(hbm_ref, row):
        def chunk(c, acc):
            pltpu.sync_copy(hbm_ref.at[pl.ds(row * _V + c * _CH, _CH)], bufx)

            def ivec(i, a):
                t = bufx[pl.ds(i * 16, 16)]
                return a + t * t

            return lax.fori_loop(0, _CVECS, ivec, acc)

        acc = lax.fori_loop(0, _NCHUNK, chunk, zero16)
        return _allsum(acc)

    def do_row(j, _):
        row = wid * _ROWS_PER_W + j

        bn2 = sum_sq_of(br_hbm, row)
        ln2 = sum_sq_of(nl_hbm, row)
        ratio2 = jnp.where(bn2 > _splat(1e-24), ln2 / bn2, _splat(1.0))
        scale16 = _sqrt16(ratio2)

        def build_chunk(c, mx):
            pltpu.sync_copy(nl_hbm.at[pl.ds(row * _V + c * _CH, _CH)], bufx)
            pltpu.sync_copy(dl_hbm.at[pl.ds(row * _V + c * _CH, _CH)], bufd)

            def ivec(i, m):
                v = bufx[pl.ds(i * 16, 16)] + scale16 * bufd[pl.ds(i * 16, 16)]
                parr[pl.ds(c * _CH + i * 16, 16)] = v
                return jnp.maximum(m, v)

            return lax.fori_loop(0, _CVECS, ivec, mx)

        m16 = _allmax(lax.fori_loop(0, _NCHUNK, build_chunk, neginf))

        def expvec(i, acc):
            v = parr[pl.ds(i * 16, 16)]
            p = jnp.exp(v - m16)
            parr[pl.ds(i * 16, 16)] = p
            return acc + p

        z = _allsum(lax.fori_loop(0, _PVECS, expvec, zero16))
        target = _splat(_TOP_P) * z

        base = _splat(0, jnp.int32)
        above = zero16
        for shift in (20, 10, 0):
            span = _splat(_K << shift, jnp.int32)
            rem = target - above

            def zvec(i, _):
                hist[pl.ds(i * 16, 16)] = zero16
                return 0

            lax.fori_loop(0, _K, zvec, 0)

            def scat(i, _):
                p = parr[pl.ds(i * 16, 16)]
                ip = lax.bitcast_convert_type(p, jnp.int32)
                off = ip - base
                inb = (off >= _splat(0, jnp.int32)) & (off < span)
                idx = ((off >> shift) << 4) + lane
                plsc.addupdate_scatter(hist, [idx], p, mask=inb)
                return 0

            lax.fori_loop(0, _PVECS, scat, 0)

            def bscan(i, carry):
                s, found, kbstar, sstar = carry
                kb = 63 - i

                def bacc(t, a):
                    return a + hist[pl.ds(kb * 256 + t * 16, 16)]

                blk = _allsum(lax.fori_loop(0, 16, bacc, zero16))
                hit = jnp.logical_and(
                    jnp.logical_not(found),
                    jnp.logical_or(s + blk > rem, i == 63))
                kbstar = jnp.where(hit, kb, kbstar)
                sstar = jnp.where(hit, s, sstar)
                return (s + blk, found | hit, kbstar, sstar)

            _, _, kbstar, sstar = lax.fori_loop(
                0, 64, bscan,
                (zero16, _splat(0, jnp.bool_), _splat(0, jnp.int32), zero16))

            kbs = kbstar[0]

            def kscan(i, carry):
                s, found, kstar, cstar = carry
                k = 15 - i
                h = _allsum(hist[pl.ds(kbs * 256 + k * 16, 16)])
                hit = jnp.logical_and(
                    jnp.logical_not(found),
                    jnp.logical_or(s + h > rem, i == 15))
                kstar = jnp.where(hit, k, kstar)
                cstar = jnp.where(hit, s, cstar)
                return (s + h, found | hit, kstar, cstar)

            _, _, kstar, cstar = lax.fori_loop(
                0, 16, kscan,
                (sstar, _splat(0, jnp.bool_), _splat(0, jnp.int32), zero16))

            kg = kbstar * 16 + kstar
            base = base + (kg << shift)
            above = above + cstar

        t16 = base

        def out_chunk(c, _):
            pltpu.sync_copy(nl_hbm.at[pl.ds(row * _V + c * _CH, _CH)], bufx)
            pltpu.sync_copy(dl_hbm.at[pl.ds(row * _V + c * _CH, _CH)], bufd)

            def ivec(i, _):
                v = bufx[pl.ds(i * 16, 16)] + scale16 * bufd[pl.ds(i * 16, 16)]
                ip = lax.bitcast_convert_type(parr[pl.ds(c * _CH + i * 16, 16)], jnp.int32)
                bufx[pl.ds(i * 16, 16)] = jnp.where(ip >= t16, v, neginf)
                return 0

            lax.fori_loop(0, _CVECS, ivec, 0)
            pltpu.sync_copy(bufx, out_hbm.at[pl.ds(row * _V + c * _CH, _CH)])
            return 0

        lax.fori_loop(0, _NCHUNK, out_chunk, 0)
        return 0

    lax.fori_loop(0, _ROWS_PER_W, do_row, 0)


def kernel(next_logits, bias_row, delta):
    mesh = plsc.VectorSubcoreMesh(core_axis_name="c", subcore_axis_name="s")
    f = functools.partial(
        pl.kernel,
        mesh=mesh,
        compiler_params=pltpu.CompilerParams(needs_layout_passes=False),
        out_type=jax.ShapeDtypeStruct((_B * _V,), jnp.float32),
        scratch_types=[
            pltpu.VMEM((_V,), jnp.float32),
            pltpu.VMEM((_K * 16,), jnp.float32),
            pltpu.VMEM((_CH,), jnp.float32),
            pltpu.VMEM((_CH,), jnp.float32),
        ],
    )(_body)
    out = f(next_logits.reshape(-1), bias_row.reshape(-1), delta.reshape(-1))
    return out.reshape(_B, _V)


# speedup vs baseline: 1.2412x; 1.2412x over previous
import functools

import jax
import jax.numpy as jnp
from jax import lax
from jax.experimental import pallas as pl
from jax.experimental.pallas import tpu as pltpu
from jax.experimental.pallas import tpu_sc as plsc

_TOP_P = 0.98
_B, _V = 128, 100000
_CH = 4000
_NCHUNK = _V // _CH
_CVECS = _CH // 16
_PVECS = _V // 16
_K = 1024
_ROWS_PER_W = 4


def _splat(x, dtype=jnp.float32):
    return jnp.broadcast_to(jnp.asarray(x, dtype), (16,))


def _ufori(n, u, body, init):
    def outer(o, carry):
        for k in range(u):
            carry = body(o * u + k, carry)
        return carry

    return lax.fori_loop(0, n // u, outer, init)


def _perm(v, idx):
    dnums = lax.GatherDimensionNumbers(
        offset_dims=(), collapsed_slice_dims=(0,), start_index_map=(0,))
    return lax.gather(v, idx[:, None], dnums, (1,),
                      unique_indices=True, indices_are_sorted=False,
                      mode=lax.GatherScatterMode.PROMISE_IN_BOUNDS)


def _allsum(v):
    lane = lax.iota(jnp.int32, 16)
    for sh in (8, 4, 2, 1):
        v = v + _perm(v, lane ^ sh)
    return v


def _allmax(v):
    lane = lax.iota(jnp.int32, 16)
    for sh in (8, 4, 2, 1):
        v = jnp.maximum(v, _perm(v, lane ^ sh))
    return v


def _sqrt16(a):
    i = lax.bitcast_convert_type(a, jnp.int32)
    y = lax.bitcast_convert_type(_splat(0x5F3759DF, jnp.int32) - (i >> 1), jnp.float32)
    half = _splat(0.5)
    three_half = _splat(1.5)
    for _ in range(4):
        y = y * (three_half - half * a * y * y)
    return a * y


def _body(nl_hbm, br_hbm, dl_hbm, out_hbm, parr, hist, bufx, bufd):
    nc = 2
    wid = lax.axis_index("s") * nc + lax.axis_index("c")
    lane = lax.iota(jnp.int32, 16)
    zero16 = _splat(0.0)
    neginf = _splat(-jnp.inf)

    def sum_sq_of(hbm_ref, row):
        def chunk(c, acc):
            pltpu.sync_copy(hbm_ref.at[pl.ds(row * _V + c * _CH, _CH)], bufx)

            def ivec(i, a):
                t = bufx[pl.ds(i * 16, 16)]
                return a + t * t

            return _ufori(_CVECS, 10, ivec, acc)

        acc = lax.fori_loop(0, _NCHUNK, chunk, zero16)
        return _allsum(acc)

    def do_row(j, _):
        row = wid * _ROWS_PER_W + j

        bn2 = sum_sq_of(br_hbm, row)
        ln2 = sum_sq_of(nl_hbm, row)
        ratio2 = jnp.where(bn2 > _splat(1e-24), ln2 / bn2, _splat(1.0))
        scale16 = _sqrt16(ratio2)

        def build_chunk(c, mx):
            pltpu.sync_copy(nl_hbm.at[pl.ds(row * _V + c * _CH, _CH)], bufx)
            pltpu.sync_copy(dl_hbm.at[pl.ds(row * _V + c * _CH, _CH)], bufd)

            def ivec(i, m):
                v = bufx[pl.ds(i * 16, 16)] + scale16 * bufd[pl.ds(i * 16, 16)]
                parr[pl.ds(c * _CH + i * 16, 16)] = v
                return jnp.maximum(m, v)

            return _ufori(_CVECS, 10, ivec, mx)

        m16 = _allmax(lax.fori_loop(0, _NCHUNK, build_chunk, neginf))

        def expvec(i, acc):
            v = parr[pl.ds(i * 16, 16)]
            p = jnp.exp(v - m16)
            parr[pl.ds(i * 16, 16)] = p
            return acc + p

        z = _allsum(_ufori(_PVECS, 10, expvec, zero16))
        target = _splat(_TOP_P) * z

        base = _splat(0, jnp.int32)
        above = zero16
        for shift in (20, 10, 0):
            span = _splat(_K << shift, jnp.int32)
            rem = target - above

            def zvec(i, _):
                hist[pl.ds(i * 16, 16)] = zero16
                return 0

            _ufori(_K, 8, zvec, 0)

            def scat(i, _):
                p = parr[pl.ds(i * 16, 16)]
                ip = lax.bitcast_convert_type(p, jnp.int32)
                off = ip - base
                inb = (off >= _splat(0, jnp.int32)) & (off < span)
                idx = ((off >> shift) << 4) + lane
                plsc.addupdate_scatter(hist, [idx], p, mask=inb)
                return 0

            _ufori(_PVECS, 10, scat, 0)

            def bscan(i, carry):
                s, found, kbstar, sstar = carry
                kb = 63 - i

                def bacc(t, a):
                    return a + hist[pl.ds(kb * 256 + t * 16, 16)]

                blk = _allsum(_ufori(16, 16, bacc, zero16))
                hit = jnp.logical_and(
                    jnp.logical_not(found),
                    jnp.logical_or(s + blk > rem, i == 63))
                kbstar = jnp.where(hit, kb, kbstar)
                sstar = jnp.where(hit, s, sstar)
                return (s + blk, found | hit, kbstar, sstar)

            _, _, kbstar, sstar = lax.fori_loop(
                0, 64, bscan,
                (zero16, _splat(0, jnp.bool_), _splat(0, jnp.int32), zero16))

            kbs = kbstar[0]

            def kscan(i, carry):
                s, found, kstar, cstar = carry
                k = 15 - i
                h = _allsum(hist[pl.ds(kbs * 256 + k * 16, 16)])
                hit = jnp.logical_and(
                    jnp.logical_not(found),
                    jnp.logical_or(s + h > rem, i == 15))
                kstar = jnp.where(hit, k, kstar)
                cstar = jnp.where(hit, s, cstar)
                return (s + h, found | hit, kstar, cstar)

            _, _, kstar, cstar = lax.fori_loop(
                0, 16, kscan,
                (sstar, _splat(0, jnp.bool_), _splat(0, jnp.int32), zero16))

            kg = kbstar * 16 + kstar
            base = base + (kg << shift)
            above = above + cstar

        t16 = base

        def out_chunk(c, _):
            pltpu.sync_copy(nl_hbm.at[pl.ds(row * _V + c * _CH, _CH)], bufx)
            pltpu.sync_copy(dl_hbm.at[pl.ds(row * _V + c * _CH, _CH)], bufd)

            def ivec(i, _):
                v = bufx[pl.ds(i * 16, 16)] + scale16 * bufd[pl.ds(i * 16, 16)]
                ip = lax.bitcast_convert_type(parr[pl.ds(c * _CH + i * 16, 16)], jnp.int32)
                bufx[pl.ds(i * 16, 16)] = jnp.where(ip >= t16, v, neginf)
                return 0

            _ufori(_CVECS, 10, ivec, 0)
            pltpu.sync_copy(bufx, out_hbm.at[pl.ds(row * _V + c * _CH, _CH)])
            return 0

        lax.fori_loop(0, _NCHUNK, out_chunk, 0)
        return 0

    lax.fori_loop(0, _ROWS_PER_W, do_row, 0)


def kernel(next_logits, bias_row, delta):
    mesh = plsc.VectorSubcoreMesh(core_axis_name="c", subcore_axis_name="s")
    f = functools.partial(
        pl.kernel,
        mesh=mesh,
        compiler_params=pltpu.CompilerParams(needs_layout_passes=False),
        out_type=jax.ShapeDtypeStruct((_B * _V,), jnp.float32),
        scratch_types=[
            pltpu.VMEM((_V,), jnp.float32),
            pltpu.VMEM((_K * 16,), jnp.float32),
            pltpu.VMEM((_CH,), jnp.float32),
            pltpu.VMEM((_CH,), jnp.float32),
        ],
    )(_body)
    out = f(next_logits.reshape(-1), bias_row.reshape(-1), delta.reshape(-1))
    return out.reshape(_B, _V)


# speedup vs baseline: 2.0927x; 1.6860x over previous
import functools

import jax
import jax.numpy as jnp
from jax import lax
from jax.experimental import pallas as pl
from jax.experimental.pallas import tpu as pltpu
from jax.experimental.pallas import tpu_sc as plsc

_TOP_P = 0.98
_B, _V = 128, 100000
_CH = 4000
_NCHUNK = _V // _CH
_CVECS = _CH // 16
_PVECS = _V // 16
_K = 1024
_ROWS_PER_W = 4


def _splat(x, dtype=jnp.float32):
    return jnp.broadcast_to(jnp.asarray(x, dtype), (16,))


def _ufori(n, u, body, init):
    def outer(o, carry):
        for k in range(u):
            carry = body(o * u + k, carry)
        return carry

    return lax.fori_loop(0, n // u, outer, init)


def _perm(v, idx):
    dnums = lax.GatherDimensionNumbers(
        offset_dims=(), collapsed_slice_dims=(0,), start_index_map=(0,))
    return lax.gather(v, idx[:, None], dnums, (1,),
                      unique_indices=True, indices_are_sorted=False,
                      mode=lax.GatherScatterMode.PROMISE_IN_BOUNDS)


def _allsum(v):
    lane = lax.iota(jnp.int32, 16)
    for sh in (8, 4, 2, 1):
        v = v + _perm(v, lane ^ sh)
    return v


def _allmax(v):
    lane = lax.iota(jnp.int32, 16)
    for sh in (8, 4, 2, 1):
        v = jnp.maximum(v, _perm(v, lane ^ sh))
    return v


def _sqrt16(a):
    i = lax.bitcast_convert_type(a, jnp.int32)
    y = lax.bitcast_convert_type(_splat(0x5F3759DF, jnp.int32) - (i >> 1), jnp.float32)
    half = _splat(0.5)
    three_half = _splat(1.5)
    for _ in range(4):
        y = y * (three_half - half * a * y * y)
    return a * y


def _body(nl_hbm, br_hbm, dl_hbm, out_hbm, parr, hist, bufx, bufd):
    nc = 2
    wid = lax.axis_index("s") * nc + lax.axis_index("c")
    lane = lax.iota(jnp.int32, 16)
    zero16 = _splat(0.0)
    neginf = _splat(-jnp.inf)

    def sum_sq_of(hbm_ref, row):
        def chunk(c, accs):
            pltpu.sync_copy(hbm_ref.at[pl.ds(row * _V + c * _CH, _CH)], bufx)

            def ivec(i, a):
                ts = [bufx[pl.ds((i + k) * 16, 16)] for k in range(5)]
                return tuple(a[k] + ts[k] * ts[k] for k in range(5))

            return plsc.parallel_loop(0, _CVECS, 5, unroll=2,
                                      carry=accs)(ivec)

        accs = lax.fori_loop(0, _NCHUNK, chunk, (zero16,) * 5)
        return _allsum(accs[0] + accs[1] + accs[2] + accs[3] + accs[4])

    def do_row(j, _):
        row = wid * _ROWS_PER_W + j

        bn2 = sum_sq_of(br_hbm, row)
        ln2 = sum_sq_of(nl_hbm, row)
        ratio2 = jnp.where(bn2 > _splat(1e-24), ln2 / bn2, _splat(1.0))
        scale16 = _sqrt16(ratio2)

        def build_chunk(c, mxs):
            pltpu.sync_copy(nl_hbm.at[pl.ds(row * _V + c * _CH, _CH)], bufx)
            pltpu.sync_copy(dl_hbm.at[pl.ds(row * _V + c * _CH, _CH)], bufd)

            def ivec(i, ms):
                out = []
                for k in range(5):
                    v = (bufx[pl.ds((i + k) * 16, 16)]
                         + scale16 * bufd[pl.ds((i + k) * 16, 16)])
                    parr[pl.ds(c * _CH + (i + k) * 16, 16)] = v
                    out.append(jnp.maximum(ms[k], v))
                return tuple(out)

            return plsc.parallel_loop(0, _CVECS, 5, unroll=2,
                                      carry=mxs)(ivec)

        mxs = lax.fori_loop(0, _NCHUNK, build_chunk, (neginf,) * 5)
        m16 = _allmax(jnp.maximum(jnp.maximum(jnp.maximum(mxs[0], mxs[1]),
                                              jnp.maximum(mxs[2], mxs[3])),
                                  mxs[4]))

        def expvec(i, accs):
            out = []
            for k in range(5):
                v = parr[pl.ds((i + k) * 16, 16)]
                pk = jnp.exp(v - m16)
                parr[pl.ds((i + k) * 16, 16)] = pk
                out.append(accs[k] + pk)
            return tuple(out)

        zaccs = plsc.parallel_loop(0, _PVECS, 5, unroll=2,
                                   carry=(zero16,) * 5)(expvec)
        z = _allsum(zaccs[0] + zaccs[1] + zaccs[2] + zaccs[3] + zaccs[4])
        target = _splat(_TOP_P) * z

        base = _splat(0, jnp.int32)
        above = zero16
        for shift in (20, 10, 0):
            span = _splat(_K << shift, jnp.int32)
            rem = target - above

            def zvec(i):
                for k in range(8):
                    hist[pl.ds((i + k) * 16, 16)] = zero16

            plsc.parallel_loop(0, _K, 8, unroll=2)(zvec)

            def scat(i):
                for k in range(5):
                    p = parr[pl.ds((i + k) * 16, 16)]
                    ip = lax.bitcast_convert_type(p, jnp.int32)
                    off = ip - base
                    inb = (off >= _splat(0, jnp.int32)) & (off < span)
                    idx = ((off >> shift) << 4) + lane
                    plsc.addupdate_scatter(hist, [idx], p, mask=inb)

            plsc.parallel_loop(0, _PVECS, 5, unroll=2)(scat)

            def bscan(i, carry):
                s, found, kbstar, sstar = carry
                kb = 63 - i

                bas = [zero16] * 4
                for t in range(16):
                    bas[t % 4] = bas[t % 4] + hist[pl.ds(kb * 256 + t * 16, 16)]
                blk = _allsum((bas[0] + bas[1]) + (bas[2] + bas[3]))
                hit = jnp.logical_and(
                    jnp.logical_not(found),
                    jnp.logical_or(s + blk > rem, i == 63))
                kbstar = jnp.where(hit, kb, kbstar)
                sstar = jnp.where(hit, s, sstar)
                return (s + blk, found | hit, kbstar, sstar)

            _, _, kbstar, sstar = lax.fori_loop(
                0, 64, bscan,
                (zero16, _splat(0, jnp.bool_), _splat(0, jnp.int32), zero16))

            kbs = kbstar[0]

            def kscan(i, carry):
                s, found, kstar, cstar = carry
                k = 15 - i
                h = _allsum(hist[pl.ds(kbs * 256 + k * 16, 16)])
                hit = jnp.logical_and(
                    jnp.logical_not(found),
                    jnp.logical_or(s + h > rem, i == 15))
                kstar = jnp.where(hit, k, kstar)
                cstar = jnp.where(hit, s, cstar)
                return (s + h, found | hit, kstar, cstar)

            _, _, kstar, cstar = lax.fori_loop(
                0, 16, kscan,
                (sstar, _splat(0, jnp.bool_), _splat(0, jnp.int32), zero16))

            kg = kbstar * 16 + kstar
            base = base + (kg << shift)
            above = above + cstar

        t16 = base

        def out_chunk(c, _):
            pltpu.sync_copy(nl_hbm.at[pl.ds(row * _V + c * _CH, _CH)], bufx)
            pltpu.sync_copy(dl_hbm.at[pl.ds(row * _V + c * _CH, _CH)], bufd)

            def ivec(i):
                for k in range(5):
                    v = (bufx[pl.ds((i + k) * 16, 16)]
                         + scale16 * bufd[pl.ds((i + k) * 16, 16)])
                    ip = lax.bitcast_convert_type(
                        parr[pl.ds(c * _CH + (i + k) * 16, 16)], jnp.int32)
                    bufx[pl.ds((i + k) * 16, 16)] = jnp.where(ip >= t16, v,
                                                              neginf)

            plsc.parallel_loop(0, _CVECS, 5, unroll=2)(ivec)
            pltpu.sync_copy(bufx, out_hbm.at[pl.ds(row * _V + c * _CH, _CH)])
            return 0

        lax.fori_loop(0, _NCHUNK, out_chunk, 0)
        return 0

    lax.fori_loop(0, _ROWS_PER_W, do_row, 0)


def kernel(next_logits, bias_row, delta):
    mesh = plsc.VectorSubcoreMesh(core_axis_name="c", subcore_axis_name="s")
    f = functools.partial(
        pl.kernel,
        mesh=mesh,
        compiler_params=pltpu.CompilerParams(needs_layout_passes=False),
        out_type=jax.ShapeDtypeStruct((_B * _V,), jnp.float32),
        scratch_types=[
            pltpu.VMEM((_V,), jnp.float32),
            pltpu.VMEM((_K * 16,), jnp.float32),
            pltpu.VMEM((_CH,), jnp.float32),
            pltpu.VMEM((_CH,), jnp.float32),
        ],
    )(_body)
    out = f(next_logits.reshape(-1), bias_row.reshape(-1), delta.reshape(-1))
    return out.reshape(_B, _V)


# speedup vs baseline: 3.0742x; 1.4690x over previous
import functools

import jax
import jax.numpy as jnp
from jax import lax
from jax.experimental import pallas as pl
from jax.experimental.pallas import tpu as pltpu
from jax.experimental.pallas import tpu_sc as plsc

_TOP_P = 0.98
_B, _V = 128, 100000
_CH = 4000
_NCHUNK = _V // _CH
_CVECS = _CH // 16
_PVECS = _V // 16
_K = 1024
_ROWS_PER_W = 4


def _splat(x, dtype=jnp.float32):
    return jnp.broadcast_to(jnp.asarray(x, dtype), (16,))


def _ufori(n, u, body, init):
    def outer(o, carry):
        for k in range(u):
            carry = body(o * u + k, carry)
        return carry

    return lax.fori_loop(0, n // u, outer, init)


def _perm(v, idx):
    dnums = lax.GatherDimensionNumbers(
        offset_dims=(), collapsed_slice_dims=(0,), start_index_map=(0,))
    return lax.gather(v, idx[:, None], dnums, (1,),
                      unique_indices=True, indices_are_sorted=False,
                      mode=lax.GatherScatterMode.PROMISE_IN_BOUNDS)


def _allsum(v):
    lane = lax.iota(jnp.int32, 16)
    for sh in (8, 4, 2, 1):
        v = v + _perm(v, lane ^ sh)
    return v


def _allmax(v):
    lane = lax.iota(jnp.int32, 16)
    for sh in (8, 4, 2, 1):
        v = jnp.maximum(v, _perm(v, lane ^ sh))
    return v


def _sqrt16(a):
    i = lax.bitcast_convert_type(a, jnp.int32)
    y = lax.bitcast_convert_type(_splat(0x5F3759DF, jnp.int32) - (i >> 1), jnp.float32)
    half = _splat(0.5)
    three_half = _splat(1.5)
    for _ in range(4):
        y = y * (three_half - half * a * y * y)
    return a * y


def _body(nl_hbm, br_hbm, dl_hbm, out_hbm, parr, hist, bufx, bufd):
    nc = 2
    wid = lax.axis_index("s") * nc + lax.axis_index("c")
    lane = lax.iota(jnp.int32, 16)
    zero16 = _splat(0.0)
    neginf = _splat(-jnp.inf)

    def sumsq_parr():
        def ivec(i, a):
            ts = [parr[pl.ds((i + k) * 16, 16)] for k in range(5)]
            return tuple(a[k] + ts[k] * ts[k] for k in range(5))

        accs = plsc.parallel_loop(0, _PVECS, 5, unroll=2,
                                  carry=(zero16,) * 5)(ivec)
        return _allsum(accs[0] + accs[1] + accs[2] + accs[3] + accs[4])

    def do_row(j, _):
        row = wid * _ROWS_PER_W + j
        rowoff = row * _V

        pltpu.sync_copy(br_hbm.at[pl.ds(rowoff, _V)], parr)
        bn2 = sumsq_parr()
        pltpu.sync_copy(nl_hbm.at[pl.ds(rowoff, _V)], parr)
        ln2 = sumsq_parr()
        ratio2 = jnp.where(bn2 > _splat(1e-24), ln2 / bn2, _splat(1.0))
        scale16 = _sqrt16(ratio2)

        def build_chunk(c, mxs):
            pltpu.sync_copy(dl_hbm.at[pl.ds(rowoff + c * _CH, _CH)], bufd)

            def ivec(i, ms):
                out = []
                for k in range(5):
                    j16 = c * _CH + (i + k) * 16
                    v = (parr[pl.ds(j16, 16)]
                         + scale16 * bufd[pl.ds((i + k) * 16, 16)])
                    parr[pl.ds(j16, 16)] = v
                    out.append(jnp.maximum(ms[k], v))
                return tuple(out)

            return plsc.parallel_loop(0, _CVECS, 5, unroll=2,
                                      carry=mxs)(ivec)

        mxs = lax.fori_loop(0, _NCHUNK, build_chunk, (neginf,) * 5)
        m16 = _allmax(jnp.maximum(jnp.maximum(jnp.maximum(mxs[0], mxs[1]),
                                              jnp.maximum(mxs[2], mxs[3])),
                                  mxs[4]))

        base = _splat(0, jnp.int32)
        above = zero16
        target = zero16
        for shift in (20, 10, 0):
            span = _splat(_K << shift, jnp.int32)

            def zvec(i):
                for k in range(8):
                    hist[pl.ds((i + k) * 16, 16)] = zero16

            plsc.parallel_loop(0, _K, 8, unroll=2)(zvec)

            def scat(i):
                for k in range(5):
                    p = jnp.exp(parr[pl.ds((i + k) * 16, 16)] - m16)
                    ip = lax.bitcast_convert_type(p, jnp.int32)
                    off = ip - base
                    inb = (off >= _splat(0, jnp.int32)) & (off < span)
                    idx = ((off >> shift) << 4) + lane
                    plsc.addupdate_scatter(hist, [idx], p, mask=inb)

            plsc.parallel_loop(0, _PVECS, 5, unroll=2)(scat)

            if shift == 20:
                def hsum(i, a):
                    return tuple(a[k] + hist[pl.ds((i + k) * 16, 16)]
                                 for k in range(8))

                hs = plsc.parallel_loop(0, _K, 8, unroll=2,
                                        carry=(zero16,) * 8)(hsum)
                z = _allsum(((hs[0] + hs[1]) + (hs[2] + hs[3]))
                            + ((hs[4] + hs[5]) + (hs[6] + hs[7])))
                target = _splat(_TOP_P) * z

            rem = target - above

            def bscan(i, carry):
                s, found, kbstar, sstar = carry
                kb = 63 - i

                bas = [zero16] * 4
                for t in range(16):
                    bas[t % 4] = bas[t % 4] + hist[pl.ds(kb * 256 + t * 16, 16)]
                blk = _allsum((bas[0] + bas[1]) + (bas[2] + bas[3]))
                hit = jnp.logical_and(
                    jnp.logical_not(found),
                    jnp.logical_or(s + blk > rem, i == 63))
                kbstar = jnp.where(hit, kb, kbstar)
                sstar = jnp.where(hit, s, sstar)
                return (s + blk, found | hit, kbstar, sstar)

            _, _, kbstar, sstar = lax.fori_loop(
                0, 64, bscan,
                (zero16, _splat(0, jnp.bool_), _splat(0, jnp.int32), zero16))

            kbs = kbstar[0]

            def kscan(i, carry):
                s, found, kstar, cstar = carry
                k = 15 - i
                h = _allsum(hist[pl.ds(kbs * 256 + k * 16, 16)])
                hit = jnp.logical_and(
                    jnp.logical_not(found),
                    jnp.logical_or(s + h > rem, i == 15))
                kstar = jnp.where(hit, k, kstar)
                cstar = jnp.where(hit, s, cstar)
                return (s + h, found | hit, kstar, cstar)

            _, _, kstar, cstar = lax.fori_loop(
                0, 16, kscan,
                (sstar, _splat(0, jnp.bool_), _splat(0, jnp.int32), zero16))

            kg = kbstar * 16 + kstar
            base = base + (kg << shift)
            above = above + cstar

        t16 = base

        def out_chunk(c, _):
            def ivec(i):
                for k in range(5):
                    v = parr[pl.ds(c * _CH + (i + k) * 16, 16)]
                    ip = lax.bitcast_convert_type(jnp.exp(v - m16), jnp.int32)
                    bufx[pl.ds((i + k) * 16, 16)] = jnp.where(ip >= t16, v,
                                                              neginf)

            plsc.parallel_loop(0, _CVECS, 5, unroll=2)(ivec)
            pltpu.sync_copy(bufx, out_hbm.at[pl.ds(rowoff + c * _CH, _CH)])
            return 0

        lax.fori_loop(0, _NCHUNK, out_chunk, 0)
        return 0

    lax.fori_loop(0, _ROWS_PER_W, do_row, 0)


def kernel(next_logits, bias_row, delta):
    mesh = plsc.VectorSubcoreMesh(core_axis_name="c", subcore_axis_name="s")
    f = functools.partial(
        pl.kernel,
        mesh=mesh,
        compiler_params=pltpu.CompilerParams(needs_layout_passes=False),
        out_type=jax.ShapeDtypeStruct((_B * _V,), jnp.float32),
        scratch_types=[
            pltpu.VMEM((_V,), jnp.float32),
            pltpu.VMEM((_K * 16,), jnp.float32),
            pltpu.VMEM((_CH,), jnp.float32),
            pltpu.VMEM((_CH,), jnp.float32),
        ],
    )(_body)
    out = f(next_logits.reshape(-1), bias_row.reshape(-1), delta.reshape(-1))
    return out.reshape(_B, _V)


# speedup vs baseline: 3.2958x; 1.0721x over previous
import functools

import jax
import jax.numpy as jnp
from jax import lax
from jax.experimental import pallas as pl
from jax.experimental.pallas import tpu as pltpu
from jax.experimental.pallas import tpu_sc as plsc

_TOP_P = 0.98
_B, _V = 128, 100000
_CH = 4000
_NCHUNK = _V // _CH
_CVECS = _CH // 16
_PVECS = _V // 16
_K = 1024
_ROWS_PER_W = 4


def _splat(x, dtype=jnp.float32):
    return jnp.broadcast_to(jnp.asarray(x, dtype), (16,))


def _ufori(n, u, body, init):
    def outer(o, carry):
        for k in range(u):
            carry = body(o * u + k, carry)
        return carry

    return lax.fori_loop(0, n // u, outer, init)


def _perm(v, idx):
    dnums = lax.GatherDimensionNumbers(
        offset_dims=(), collapsed_slice_dims=(0,), start_index_map=(0,))
    return lax.gather(v, idx[:, None], dnums, (1,),
                      unique_indices=True, indices_are_sorted=False,
                      mode=lax.GatherScatterMode.PROMISE_IN_BOUNDS)


def _allsum(v):
    lane = lax.iota(jnp.int32, 16)
    for sh in (8, 4, 2, 1):
        v = v + _perm(v, lane ^ sh)
    return v


def _allmax(v):
    lane = lax.iota(jnp.int32, 16)
    for sh in (8, 4, 2, 1):
        v = jnp.maximum(v, _perm(v, lane ^ sh))
    return v


def _sqrt16(a):
    i = lax.bitcast_convert_type(a, jnp.int32)
    y = lax.bitcast_convert_type(_splat(0x5F3759DF, jnp.int32) - (i >> 1), jnp.float32)
    half = _splat(0.5)
    three_half = _splat(1.5)
    for _ in range(4):
        y = y * (three_half - half * a * y * y)
    return a * y


def _body(nl_hbm, br_hbm, dl_hbm, out_hbm, parr, hist, bufx, bufd):
    nc = 2
    wid = lax.axis_index("s") * nc + lax.axis_index("c")
    lane = lax.iota(jnp.int32, 16)
    zero16 = _splat(0.0)
    neginf = _splat(-jnp.inf)

    def sumsq_parr():
        def ivec(i, a):
            ts = [parr[pl.ds((i + k) * 16, 16)] for k in range(5)]
            return tuple(a[k] + ts[k] * ts[k] for k in range(5))

        accs = plsc.parallel_loop(0, _PVECS, 5, unroll=2,
                                  carry=(zero16,) * 5)(ivec)
        return _allsum(accs[0] + accs[1] + accs[2] + accs[3] + accs[4])

    def do_row(j, _):
        row = wid * _ROWS_PER_W + j
        rowoff = row * _V

        pltpu.sync_copy(br_hbm.at[pl.ds(rowoff, _V)], parr)
        bn2 = sumsq_parr()
        pltpu.sync_copy(nl_hbm.at[pl.ds(rowoff, _V)], parr)
        ln2 = sumsq_parr()
        ratio2 = jnp.where(bn2 > _splat(1e-24), ln2 / bn2, _splat(1.0))
        scale16 = _sqrt16(ratio2)

        def build_chunk(c, mxs):
            pltpu.sync_copy(dl_hbm.at[pl.ds(rowoff + c * _CH, _CH)], bufd)

            def ivec(i, ms):
                out = []
                for k in range(5):
                    j16 = c * _CH + (i + k) * 16
                    v = (parr[pl.ds(j16, 16)]
                         + scale16 * bufd[pl.ds((i + k) * 16, 16)])
                    parr[pl.ds(j16, 16)] = v
                    out.append(jnp.maximum(ms[k], v))
                return tuple(out)

            return plsc.parallel_loop(0, _CVECS, 5, unroll=2,
                                      carry=mxs)(ivec)

        mxs = lax.fori_loop(0, _NCHUNK, build_chunk, (neginf,) * 5)
        m16 = _allmax(jnp.maximum(jnp.maximum(jnp.maximum(mxs[0], mxs[1]),
                                              jnp.maximum(mxs[2], mxs[3])),
                                  mxs[4]))

        base = _splat(0, jnp.int32)
        above = zero16
        target = zero16
        for shift in (20, 10):
            span = _splat(_K << shift, jnp.int32)

            def zvec(i):
                for k in range(8):
                    hist[pl.ds((i + k) * 16, 16)] = zero16

            plsc.parallel_loop(0, _K, 8, unroll=2)(zvec)

            def scat(i):
                for k in range(5):
                    p = jnp.exp(parr[pl.ds((i + k) * 16, 16)] - m16)
                    ip = lax.bitcast_convert_type(p, jnp.int32)
                    off = ip - base
                    inb = (off >= _splat(0, jnp.int32)) & (off < span)
                    idx = ((off >> shift) << 4) + lane
                    plsc.addupdate_scatter(hist, [idx], p, mask=inb)

            plsc.parallel_loop(0, _PVECS, 5, unroll=2)(scat)

            if shift == 20:
                def hsum(i, a):
                    return tuple(a[k] + hist[pl.ds((i + k) * 16, 16)]
                                 for k in range(8))

                hs = plsc.parallel_loop(0, _K, 8, unroll=2,
                                        carry=(zero16,) * 8)(hsum)
                z = _allsum(((hs[0] + hs[1]) + (hs[2] + hs[3]))
                            + ((hs[4] + hs[5]) + (hs[6] + hs[7])))
                target = _splat(_TOP_P) * z

            rem = target - above

            def bscan(i, carry):
                s, found, kbstar, sstar = carry
                kb = 63 - i

                bas = [zero16] * 4
                for t in range(16):
                    bas[t % 4] = bas[t % 4] + hist[pl.ds(kb * 256 + t * 16, 16)]
                blk = _allsum((bas[0] + bas[1]) + (bas[2] + bas[3]))
                hit = jnp.logical_and(
                    jnp.logical_not(found),
                    jnp.logical_or(s + blk > rem, i == 63))
                kbstar = jnp.where(hit, kb, kbstar)
                sstar = jnp.where(hit, s, sstar)
                return (s + blk, found | hit, kbstar, sstar)

            _, _, kbstar, sstar = lax.fori_loop(
                0, 64, bscan,
                (zero16, _splat(0, jnp.bool_), _splat(0, jnp.int32), zero16))

            kbs = kbstar[0]

            def kscan(i, carry):
                s, found, kstar, cstar = carry
                k = 15 - i
                h = _allsum(hist[pl.ds(kbs * 256 + k * 16, 16)])
                hit = jnp.logical_and(
                    jnp.logical_not(found),
                    jnp.logical_or(s + h > rem, i == 15))
                kstar = jnp.where(hit, k, kstar)
                cstar = jnp.where(hit, s, cstar)
                return (s + h, found | hit, kstar, cstar)

            _, _, kstar, cstar = lax.fori_loop(
                0, 16, kscan,
                (sstar, _splat(0, jnp.bool_), _splat(0, jnp.int32), zero16))

            kg = kbstar * 16 + kstar
            base = base + (kg << shift)
            above = above + cstar

        t16 = base

        def dbis(_, carry):
            ulo, uhi = carry
            mid = ulo + ((uhi - ulo) >> 1)
            d = lax.bitcast_convert_type(mid, jnp.float32)
            cond = lax.bitcast_convert_type(jnp.exp(d), jnp.int32) >= t16
            return (jnp.where(cond, mid, ulo), jnp.where(cond, uhi, mid))

        ulo0 = _splat(-0x80000000, jnp.int32)
        uhi0 = lax.bitcast_convert_type(_splat(-200.0), jnp.int32)
        ulo, _ = lax.fori_loop(0, 32, dbis, (ulo0, uhi0))
        dt16 = lax.bitcast_convert_type(ulo, jnp.float32)

        def out_chunk(c, _):
            def ivec(i):
                for k in range(5):
                    v = parr[pl.ds(c * _CH + (i + k) * 16, 16)]
                    bufx[pl.ds((i + k) * 16, 16)] = jnp.where(
                        v - m16 >= dt16, v, neginf)

            plsc.parallel_loop(0, _CVECS, 5, unroll=2)(ivec)
            pltpu.sync_copy(bufx, out_hbm.at[pl.ds(rowoff + c * _CH, _CH)])
            return 0

        lax.fori_loop(0, _NCHUNK, out_chunk, 0)
        return 0

    lax.fori_loop(0, _ROWS_PER_W, do_row, 0)


def kernel(next_logits, bias_row, delta):
    mesh = plsc.VectorSubcoreMesh(core_axis_name="c", subcore_axis_name="s")
    f = functools.partial(
        pl.kernel,
        mesh=mesh,
        compiler_params=pltpu.CompilerParams(needs_layout_passes=False),
        out_type=jax.ShapeDtypeStruct((_B * _V,), jnp.float32),
        scratch_types=[
            pltpu.VMEM((_V,), jnp.float32),
            pltpu.VMEM((_K * 16,), jnp.float32),
            pltpu.VMEM((_CH,), jnp.float32),
            pltpu.VMEM((_CH,), jnp.float32),
        ],
    )(_body)
    out = f(next_logits.reshape(-1), bias_row.reshape(-1), delta.reshape(-1))
    return out.reshape(_B, _V)


# speedup vs baseline: 3.3429x; 1.0143x over previous
import functools

import jax
import jax.numpy as jnp
from jax import lax
from jax.experimental import pallas as pl
from jax.experimental.pallas import tpu as pltpu
from jax.experimental.pallas import tpu_sc as plsc

_TOP_P = 0.98
_B, _V = 128, 100000
_CH = 4000
_NCHUNK = _V // _CH
_NPAIR = (_NCHUNK - 1) // 2
_CVECS = _CH // 16
_PVECS = _V // 16
_K = 1024
_ROWS_PER_W = 4


def _splat(x, dtype=jnp.float32):
    return jnp.broadcast_to(jnp.asarray(x, dtype), (16,))


def _perm(v, idx):
    dnums = lax.GatherDimensionNumbers(
        offset_dims=(), collapsed_slice_dims=(0,), start_index_map=(0,))
    return lax.gather(v, idx[:, None], dnums, (1,),
                      unique_indices=True, indices_are_sorted=False,
                      mode=lax.GatherScatterMode.PROMISE_IN_BOUNDS)


def _allsum(v):
    lane = lax.iota(jnp.int32, 16)
    for sh in (8, 4, 2, 1):
        v = v + _perm(v, lane ^ sh)
    return v


def _allmax(v):
    lane = lax.iota(jnp.int32, 16)
    for sh in (8, 4, 2, 1):
        v = jnp.maximum(v, _perm(v, lane ^ sh))
    return v


def _sqrt16(a):
    i = lax.bitcast_convert_type(a, jnp.int32)
    y = lax.bitcast_convert_type(_splat(0x5F3759DF, jnp.int32) - (i >> 1),
                                 jnp.float32)
    half = _splat(0.5)
    three_half = _splat(1.5)
    for _ in range(4):
        y = y * (three_half - half * a * y * y)
    return a * y


def _body(nl_hbm, br_hbm, dl_hbm, out_hbm, parr, hist, bufx, bufd,
          sem_x, sem_a, sem_b):
    nc = 2
    wid = lax.axis_index("s") * nc + lax.axis_index("c")
    lane = lax.iota(jnp.int32, 16)
    zero16 = _splat(0.0)
    neginf = _splat(-jnp.inf)

    def sumsq_buf(buf, accs):
        def ivec(i, a):
            ts = [buf[pl.ds((i + k) * 16, 16)] for k in range(5)]
            return tuple(a[k] + ts[k] * ts[k] for k in range(5))

        return plsc.parallel_loop(0, _CVECS, 5, unroll=2, carry=accs)(ivec)

    def do_row(j, _):
        row = wid * _ROWS_PER_W + j
        rowoff = row * _V

        cpx = pltpu.async_copy(nl_hbm.at[pl.ds(rowoff, _V)], parr, sem_x)

        pltpu.async_copy(br_hbm.at[pl.ds(rowoff, _CH)], bufx, sem_a)

        def bias_pair(g, accs):
            c0 = 2 * g
            pltpu.make_async_copy(
                br_hbm.at[pl.ds(rowoff + c0 * _CH, _CH)], bufx, sem_a).wait()
            pltpu.async_copy(
                br_hbm.at[pl.ds(rowoff + (c0 + 1) * _CH, _CH)], bufd, sem_b)
            accs = sumsq_buf(bufx, accs)
            pltpu.make_async_copy(
                br_hbm.at[pl.ds(rowoff + (c0 + 1) * _CH, _CH)], bufd,
                sem_b).wait()
            pltpu.async_copy(
                br_hbm.at[pl.ds(rowoff + (c0 + 2) * _CH, _CH)], bufx, sem_a)
            return sumsq_buf(bufd, accs)

        accs = lax.fori_loop(0, _NPAIR, bias_pair, (zero16,) * 5)
        pltpu.make_async_copy(
            br_hbm.at[pl.ds(rowoff + (_NCHUNK - 1) * _CH, _CH)], bufx,
            sem_a).wait()
        accs = sumsq_buf(bufx, accs)
        bn2 = _allsum(accs[0] + accs[1] + accs[2] + accs[3] + accs[4])

        cpx.wait()

        def xvec(i, a):
            ts = [parr[pl.ds((i + k) * 16, 16)] for k in range(5)]
            return tuple(a[k] + ts[k] * ts[k] for k in range(5))

        xaccs = plsc.parallel_loop(0, _PVECS, 5, unroll=2,
                                   carry=(zero16,) * 5)(xvec)
        ln2 = _allsum(xaccs[0] + xaccs[1] + xaccs[2] + xaccs[3] + xaccs[4])
        ratio2 = jnp.where(bn2 > _splat(1e-24), ln2 / bn2, _splat(1.0))
        scale16 = _sqrt16(ratio2)

        def bpass(buf, c, ms):
            def ivec(i, ms_):
                out = []
                for k in range(5):
                    j16 = c * _CH + (i + k) * 16
                    v = (parr[pl.ds(j16, 16)]
                         + scale16 * buf[pl.ds((i + k) * 16, 16)])
                    parr[pl.ds(j16, 16)] = v
                    out.append(jnp.maximum(ms_[k], v))
                return tuple(out)

            return plsc.parallel_loop(0, _CVECS, 5, unroll=2,
                                      carry=ms)(ivec)

        pltpu.async_copy(dl_hbm.at[pl.ds(rowoff, _CH)], bufx, sem_a)

        def build_pair(g, ms):
            c0 = 2 * g
            pltpu.make_async_copy(
                dl_hbm.at[pl.ds(rowoff + c0 * _CH, _CH)], bufx, sem_a).wait()
            pltpu.async_copy(
                dl_hbm.at[pl.ds(rowoff + (c0 + 1) * _CH, _CH)], bufd, sem_b)
            ms = bpass(bufx, c0, ms)
            pltpu.make_async_copy(
                dl_hbm.at[pl.ds(rowoff + (c0 + 1) * _CH, _CH)], bufd,
                sem_b).wait()
            pltpu.async_copy(
                dl_hbm.at[pl.ds(rowoff + (c0 + 2) * _CH, _CH)], bufx, sem_a)
            return bpass(bufd, c0 + 1, ms)

        mxs = lax.fori_loop(0, _NPAIR, build_pair, (neginf,) * 5)
        pltpu.make_async_copy(
            dl_hbm.at[pl.ds(rowoff + (_NCHUNK - 1) * _CH, _CH)], bufx,
            sem_a).wait()
        mxs = bpass(bufx, _NCHUNK - 1, mxs)
        m16 = _allmax(jnp.maximum(jnp.maximum(jnp.maximum(mxs[0], mxs[1]),
                                              jnp.maximum(mxs[2], mxs[3])),
                                  mxs[4]))

        base = _splat(0, jnp.int32)
        above = zero16
        target = zero16
        for shift in (20, 10):
            span = _splat(_K << shift, jnp.int32)

            def zvec(i):
                for k in range(8):
                    hist[pl.ds((i + k) * 16, 16)] = zero16

            plsc.parallel_loop(0, _K, 8, unroll=2)(zvec)

            def scat(i):
                for k in range(5):
                    p = jnp.exp(parr[pl.ds((i + k) * 16, 16)] - m16)
                    ip = lax.bitcast_convert_type(p, jnp.int32)
                    off = ip - base
                    inb = (off >= _splat(0, jnp.int32)) & (off < span)
                    idx = ((off >> shift) << 4) + lane
                    plsc.addupdate_scatter(hist, [idx], p, mask=inb)

            plsc.parallel_loop(0, _PVECS, 5, unroll=2)(scat)

            if shift == 20:
                def hsum(i, a):
                    return tuple(a[k] + hist[pl.ds((i + k) * 16, 16)]
                                 for k in range(8))

                hs = plsc.parallel_loop(0, _K, 8, unroll=2,
                                        carry=(zero16,) * 8)(hsum)
                z = _allsum(((hs[0] + hs[1]) + (hs[2] + hs[3]))
                            + ((hs[4] + hs[5]) + (hs[6] + hs[7])))
                target = _splat(_TOP_P) * z

            rem = target - above

            def bscan(i, carry):
                s, found, kbstar, sstar = carry
                kb = 63 - i

                bas = [zero16] * 4
                for t in range(16):
                    bas[t % 4] = bas[t % 4] + hist[pl.ds(kb * 256 + t * 16,
                                                         16)]
                blk = _allsum((bas[0] + bas[1]) + (bas[2] + bas[3]))
                hit = jnp.logical_and(
                    jnp.logical_not(found),
                    jnp.logical_or(s + blk > rem, i == 63))
                kbstar = jnp.where(hit, kb, kbstar)
                sstar = jnp.where(hit, s, sstar)
                return (s + blk, found | hit, kbstar, sstar)

            _, _, kbstar, sstar = lax.fori_loop(
                0, 64, bscan,
                (zero16, _splat(0, jnp.bool_), _splat(0, jnp.int32), zero16))

            kbs = kbstar[0]

            def kscan(i, carry):
                s, found, kstar, cstar = carry
                k = 15 - i
                h = _allsum(hist[pl.ds(kbs * 256 + k * 16, 16)])
                hit = jnp.logical_and(
                    jnp.logical_not(found),
                    jnp.logical_or(s + h > rem, i == 15))
                kstar = jnp.where(hit, k, kstar)
                cstar = jnp.where(hit, s, cstar)
                return (s + h, found | hit, kstar, cstar)

            _, _, kstar, cstar = lax.fori_loop(
                0, 16, kscan,
                (sstar, _splat(0, jnp.bool_), _splat(0, jnp.int32), zero16))

            kg = kbstar * 16 + kstar
            base = base + (kg << shift)
            above = above + cstar

        t16 = base

        def dbis(_, carry):
            ulo, uhi = carry
            mid = ulo + ((uhi - ulo) >> 1)
            d = lax.bitcast_convert_type(mid, jnp.float32)
            cond = lax.bitcast_convert_type(jnp.exp(d), jnp.int32) >= t16
            return (jnp.where(cond, mid, ulo), jnp.where(cond, uhi, mid))

        ulo0 = _splat(-0x80000000, jnp.int32)
        uhi0 = lax.bitcast_convert_type(_splat(-200.0), jnp.int32)
        ulo, _ = lax.fori_loop(0, 32, dbis, (ulo0, uhi0))
        dt16 = lax.bitcast_convert_type(ulo, jnp.float32)

        def fcomp(buf, c):
            def ivec(i):
                for k in range(5):
                    v = parr[pl.ds(c * _CH + (i + k) * 16, 16)]
                    buf[pl.ds((i + k) * 16, 16)] = jnp.where(
                        v - m16 >= dt16, v, neginf)

            plsc.parallel_loop(0, _CVECS, 5, unroll=2)(ivec)

        def out_pair(g, _):
            c0 = 2 * g

            @pl.when(g > 0)
            def _wa():
                pltpu.make_async_copy(
                    bufx, out_hbm.at[pl.ds(rowoff + (c0 - 2) * _CH, _CH)],
                    sem_a).wait()

            fcomp(bufx, c0)
            pltpu.async_copy(bufx, out_hbm.at[pl.ds(rowoff + c0 * _CH, _CH)],
                             sem_a)

            @pl.when(g > 0)
            def _wb():
                pltpu.make_async_copy(
                    bufd, out_hbm.at[pl.ds(rowoff + (c0 - 1) * _CH, _CH)],
                    sem_b).wait()

            fcomp(bufd, c0 + 1)
            pltpu.async_copy(bufd,
                             out_hbm.at[pl.ds(rowoff + (c0 + 1) * _CH, _CH)],
                             sem_b)
            return 0

        lax.fori_loop(0, _NPAIR, out_pair, 0)
        last = _NCHUNK - 1
        pltpu.make_async_copy(
            bufx, out_hbm.at[pl.ds(rowoff + (last - 2) * _CH, _CH)],
            sem_a).wait()
        fcomp(bufx, last)
        pltpu.async_copy(bufx, out_hbm.at[pl.ds(rowoff + last * _CH, _CH)],
                         sem_a)
        pltpu.make_async_copy(
            bufx, out_hbm.at[pl.ds(rowoff + last * _CH, _CH)], sem_a).wait()
        pltpu.make_async_copy(
            bufd, out_hbm.at[pl.ds(rowoff + (last - 1) * _CH, _CH)],
            sem_b).wait()
        return 0

    lax.fori_loop(0, _ROWS_PER_W, do_row, 0)


def kernel(next_logits, bias_row, delta):
    mesh = plsc.VectorSubcoreMesh(core_axis_name="c", subcore_axis_name="s")
    f = functools.partial(
        pl.kernel,
        mesh=mesh,
        compiler_params=pltpu.CompilerParams(needs_layout_passes=False),
        out_type=jax.ShapeDtypeStruct((_B * _V,), jnp.float32),
        scratch_types=[
            pltpu.VMEM((_V,), jnp.float32),
            pltpu.VMEM((_K * 16,), jnp.float32),
            pltpu.VMEM((_CH,), jnp.float32),
            pltpu.VMEM((_CH,), jnp.float32),
            pltpu.SemaphoreType.DMA,
            pltpu.SemaphoreType.DMA,
            pltpu.SemaphoreType.DMA,
        ],
    )(_body)
    out = f(next_logits.reshape(-1), bias_row.reshape(-1), delta.reshape(-1))
    return out.reshape(_B, _V)
